# baseline (speedup 1.0000x reference)
import functools

import numpy as np
import jax
import jax.numpy as jnp
from jax import lax
from jax.experimental import pallas as pl
from jax.experimental.pallas import tpu as pltpu
from jax.experimental.pallas import tpu_sc as plsc
from jax.experimental.layout import Format, Layout

VOCAB = 1000000
D_MODEL = 64
SEQ_LEN = 200

_NC = 2
_NS = 16
_B_PER_W = 8
_N_ACTIVE = SEQ_LEN // _B_PER_W
_NBUF = 4
_LAST_FULL_BASE = (VOCAB // 128 - 1) * 128
_TAIL_BASE = (VOCAB // 128) * 128
_TAIL_W = VOCAB - _TAIL_BASE


def _pe_table():
    pos = np.arange(SEQ_LEN, dtype=np.float32)[:, None]
    i = np.arange(D_MODEL, dtype=np.float32)[None, :]
    angle_rates = 1.0 / np.power(10000.0, (2.0 * np.floor(i / 2.0)) / D_MODEL)
    angles = pos * angle_rates
    pe = np.zeros((SEQ_LEN, D_MODEL), dtype=np.float32)
    pe[:, 0::2] = np.sin(angles[:, 0::2])
    pe[:, 1::2] = np.cos(angles[:, 1::2])
    return pe


_PE_FLAT = np.ascontiguousarray(_pe_table().reshape(-1))


def _sc_body(tt_hbm, idx_hbm, pe_hbm, out_hbm,
             idx_v, slabs_v, tail_v, rows_v, pe_v,
             sem0, sem1, sem2, sem3, sem_pe, sem_tail):
    wid = lax.axis_index("s") * _NC + lax.axis_index("c")
    sems = (sem0, sem1, sem2, sem3)

    @pl.when(wid < _N_ACTIVE)
    def _():
        base = wid * _B_PER_W
        pltpu.sync_copy(idx_hbm.at[pl.ds(base, _B_PER_W)],
                        idx_v.at[pl.ds(0, _B_PER_W)])
        iv = idx_v[...]
        lane_bases = jnp.minimum((iv >> 7) << 7, _LAST_FULL_BASE)
        cols = jnp.minimum(iv - lane_bases, 127)
        tcols = jnp.clip(iv - _TAIL_BASE, 0, _TAIL_W - 1)

        pe_copy = pltpu.async_copy(
            pe_hbm.at[pl.ds(base * D_MODEL, _B_PER_W * D_MODEL)], pe_v,
            sem_pe)
        tail_copy = pltpu.async_copy(
            tt_hbm.at[:, pl.ds(_TAIL_BASE, _TAIL_W)], tail_v, sem_tail)

        def fire(r):
            b = r % _NBUF
            return pltpu.async_copy(
                tt_hbm.at[:, pl.ds(pl.multiple_of(lane_bases[r], 128), 128)],
                slabs_v.at[b], sems[b])

        fetch = [None] * _NBUF
        for r in range(_NBUF - 1):
            fetch[r % _NBUF] = fire(r)
        pe_copy.wait()
        for r in range(_B_PER_W):
            b = r % _NBUF
            nxt = r + _NBUF - 1
            if nxt < _B_PER_W:
                fetch[nxt % _NBUF] = fire(nxt)
            fetch[b].wait()
            col = jnp.full((16,), cols[r], dtype=jnp.int32)
            for k in range(0, D_MODEL, 16):
                d_idx = lax.iota(jnp.int32, 16) + k
                gathered = plsc.load_gather(slabs_v.at[b], [d_idx, col])
                rows_v[r, pl.ds(k, 16)] = (
                    gathered + pe_v[pl.ds(r * D_MODEL + k, 16)]
                )
        tail_copy.wait()
        for r in range(_B_PER_W):
            @pl.when(iv[r] >= _TAIL_BASE)
            def _fix(r=r):
                tcol = jnp.full((16,), tcols[r], dtype=jnp.int32)
                for k in range(0, D_MODEL, 16):
                    d_idx = lax.iota(jnp.int32, 16) + k
                    gathered = plsc.load_gather(tail_v, [d_idx, tcol])
                    rows_v[r, pl.ds(k, 16)] = (
                        gathered + pe_v[pl.ds(r * D_MODEL + k, 16)]
                    )
        pltpu.sync_copy(rows_v, out_hbm.at[0, pl.ds(base, _B_PER_W)])


def _kernel_impl(x, table, pe):
    tt = jnp.swapaxes(table, 0, 1)
    mesh = plsc.VectorSubcoreMesh(core_axis_name="c", subcore_axis_name="s")
    k = pl.kernel(
        _sc_body,
        mesh=mesh,
        out_type=jax.ShapeDtypeStruct((1, SEQ_LEN, D_MODEL), jnp.float32),
        scratch_types=[
            pltpu.VMEM((16,), jnp.int32),
            pltpu.VMEM((_NBUF, D_MODEL, 128), jnp.float32),
            pltpu.VMEM((D_MODEL, _TAIL_W), jnp.float32),
            pltpu.VMEM((_B_PER_W, D_MODEL), jnp.float32),
            pltpu.VMEM((_B_PER_W * D_MODEL,), jnp.float32),
            pltpu.SemaphoreType.DMA,
            pltpu.SemaphoreType.DMA,
            pltpu.SemaphoreType.DMA,
            pltpu.SemaphoreType.DMA,
            pltpu.SemaphoreType.DMA,
            pltpu.SemaphoreType.DMA,
        ],
        compiler_params=pltpu.CompilerParams(needs_layout_passes=False),
    )
    return k(tt, x.astype(jnp.int32), pe)


_DEFAULT_JIT = jax.jit(_kernel_impl)
_STATE_CACHE = {}


def kernel(x, table):
    try:
        dev = next(iter(table.devices()))
    except (AttributeError, TypeError):
        return _DEFAULT_JIT(x, table, jnp.asarray(_PE_FLAT))
    state = _STATE_CACHE.get(dev)
    if state is None:
        fmt = Format(Layout(major_to_minor=(0, 1, 2)),
                     jax.sharding.SingleDeviceSharding(dev))
        fn = jax.jit(_kernel_impl, out_shardings=fmt)
        pe_dev = jax.device_put(_PE_FLAT, dev)
        state = _STATE_CACHE[dev] = (fn, pe_dev)
    fn, pe_dev = state
    return fn(x, table, pe_dev)


# speedup vs baseline: 1.0245x; 1.0245x over previous
import functools

import numpy as np
import jax
import jax.numpy as jnp
from jax import lax
from jax.experimental import pallas as pl
from jax.experimental.pallas import tpu as pltpu
from jax.experimental.pallas import tpu_sc as plsc
VOCAB = 1000000
D_MODEL = 64
SEQ_LEN = 200

_NC = 2
_NS = 16
_B_PER_W = 8
_N_ACTIVE = SEQ_LEN // _B_PER_W
_NBUF = 8
_LAST_FULL_BASE = (VOCAB // 128 - 1) * 128
_TAIL_BASE = (VOCAB // 128) * 128
_TAIL_W = VOCAB - _TAIL_BASE


def _pe_table():
    pos = np.arange(SEQ_LEN, dtype=np.float32)[:, None]
    i = np.arange(D_MODEL, dtype=np.float32)[None, :]
    angle_rates = 1.0 / np.power(10000.0, (2.0 * np.floor(i / 2.0)) / D_MODEL)
    angles = pos * angle_rates
    pe = np.zeros((SEQ_LEN, D_MODEL), dtype=np.float32)
    pe[:, 0::2] = np.sin(angles[:, 0::2])
    pe[:, 1::2] = np.cos(angles[:, 1::2])
    return pe


_PE_FLAT = np.ascontiguousarray(_pe_table().reshape(-1))


def _sc_body(tt_hbm, idx_hbm, pe_hbm, out_hbm,
             idx_v, slabs_v, tail_v, rows_v, pe_v,
             sem0, sem1, sem2, sem3, sem4, sem5, sem6, sem7,
             sem_pe, sem_tail):
    wid = lax.axis_index("s") * _NC + lax.axis_index("c")
    sems = (sem0, sem1, sem2, sem3, sem4, sem5, sem6, sem7)

    @pl.when(wid < _N_ACTIVE)
    def _():
        base = wid * _B_PER_W
        pltpu.sync_copy(idx_hbm.at[pl.ds(base, _B_PER_W)],
                        idx_v.at[pl.ds(0, _B_PER_W)])
        iv = idx_v[...]
        lane_bases = jnp.minimum((iv >> 7) << 7, _LAST_FULL_BASE)
        cols = jnp.minimum(iv - lane_bases, 127)
        tcols = jnp.clip(iv - _TAIL_BASE, 0, _TAIL_W - 1)

        pe_copy = pltpu.async_copy(
            pe_hbm.at[pl.ds(base * D_MODEL, _B_PER_W * D_MODEL)], pe_v,
            sem_pe)
        tail_copy = pltpu.async_copy(
            tt_hbm.at[:, pl.ds(_TAIL_BASE, _TAIL_W)], tail_v, sem_tail)

        def fire(r):
            b = r % _NBUF
            return pltpu.async_copy(
                tt_hbm.at[:, pl.ds(pl.multiple_of(lane_bases[r], 128), 128)],
                slabs_v.at[b], sems[b])

        fetch = [fire(r) for r in range(_B_PER_W)]
        pe_copy.wait()
        for r in range(_B_PER_W):
            b = r % _NBUF
            fetch[r].wait()
            col = jnp.full((16,), cols[r], dtype=jnp.int32)
            for k in range(0, D_MODEL, 16):
                d_idx = lax.iota(jnp.int32, 16) + k
                gathered = plsc.load_gather(slabs_v.at[b], [d_idx, col])
                rows_v[r, pl.ds(k, 16)] = (
                    gathered + pe_v[pl.ds(r * D_MODEL + k, 16)]
                )
        tail_copy.wait()
        for r in range(_B_PER_W):
            @pl.when(iv[r] >= _TAIL_BASE)
            def _fix(r=r):
                tcol = jnp.full((16,), tcols[r], dtype=jnp.int32)
                for k in range(0, D_MODEL, 16):
                    d_idx = lax.iota(jnp.int32, 16) + k
                    gathered = plsc.load_gather(tail_v, [d_idx, tcol])
                    rows_v[r, pl.ds(k, 16)] = (
                        gathered + pe_v[pl.ds(r * D_MODEL + k, 16)]
                    )
        pltpu.sync_copy(rows_v, out_hbm.at[0, pl.ds(base, _B_PER_W)])


def _kernel_impl(x, table, pe):
    tt = jnp.swapaxes(table, 0, 1)
    mesh = plsc.VectorSubcoreMesh(core_axis_name="c", subcore_axis_name="s")
    k = pl.kernel(
        _sc_body,
        mesh=mesh,
        out_type=jax.ShapeDtypeStruct((1, SEQ_LEN, D_MODEL), jnp.float32),
        scratch_types=[
            pltpu.VMEM((16,), jnp.int32),
            pltpu.VMEM((_NBUF, D_MODEL, 128), jnp.float32),
            pltpu.VMEM((D_MODEL, _TAIL_W), jnp.float32),
            pltpu.VMEM((_B_PER_W, D_MODEL), jnp.float32),
            pltpu.VMEM((_B_PER_W * D_MODEL,), jnp.float32),
        ] + [pltpu.SemaphoreType.DMA] * 10,
        compiler_params=pltpu.CompilerParams(needs_layout_passes=False),
    )
    return k(tt, x.astype(jnp.int32), pe)


def kernel(x, table):
    return _kernel_impl(x, table, jnp.asarray(_PE_FLAT))
